# Initial kernel scaffold; baseline (speedup 1.0000x reference)
import functools

import jax
import jax.numpy as jnp
from jax import lax
from jax.experimental import pallas as pl
from jax.experimental.pallas import tpu as pltpu
from jax.experimental.pallas import tpu_sc as plsc

NNODES = 10000
NPAD = 10240
NEDGES = 320000
NC = 2
NS = 16
NW = NC * NS
CHUNK = 128
CPW = 80
EPAD = NW * CPW * CHUNK
RPS = NPAD // NS
EPS = 1e-5


def _make_sc_degree():
  mesh = plsc.VectorSubcoreMesh(core_axis_name="c", subcore_axis_name="s")

  @functools.partial(
      pl.kernel,
      out_type=jax.ShapeDtypeStruct((NC, NPAD, 8), jnp.float32),
      mesh=mesh,
      scratch_types=[
          pltpu.VMEM((CPW, CHUNK), jnp.int32),
          pltpu.VMEM((CHUNK, 8), jnp.float32),
          pltpu.VMEM_SHARED((NPAD, 8), jnp.float32),
      ],
  )
  def k(dst_hbm, ones_hbm, zeros_hbm, out_hbm, dst_v, ones_v, acc_sh):
    c = lax.axis_index("c")
    s = lax.axis_index("s")
    w = c * NS + s
    pltpu.sync_copy(zeros_hbm.at[pl.ds(s * RPS, RPS)],
                    acc_sh.at[pl.ds(s * RPS, RPS)])
    pltpu.sync_copy(dst_hbm.at[pl.ds(w * CPW, CPW)], dst_v)
    pltpu.sync_copy(ones_hbm, ones_v)
    plsc.subcore_barrier()

    def body(j, carry):
      pltpu.sync_copy(ones_v, acc_sh.at[dst_v.at[j]], add=True)
      return carry

    lax.fori_loop(0, CPW, body, 0)
    plsc.subcore_barrier()
    pltpu.sync_copy(acc_sh.at[pl.ds(s * RPS, RPS)],
                    out_hbm.at[c, pl.ds(s * RPS, RPS)])

  return k


def _make_sc_agg(feat):
  mesh = plsc.VectorSubcoreMesh(core_axis_name="c", subcore_axis_name="s")

  @functools.partial(
      pl.kernel,
      out_type=jax.ShapeDtypeStruct((NC, NPAD, feat), jnp.float32),
      mesh=mesh,
      scratch_types=[
          pltpu.VMEM((CPW, CHUNK), jnp.int32),
          pltpu.VMEM((CPW, CHUNK), jnp.int32),
          pltpu.VMEM((CHUNK, feat), jnp.float32),
          pltpu.VMEM_SHARED((NPAD, feat), jnp.float32),
          pltpu.SemaphoreType.DMA,
      ],
  )
  def k(ht_hbm, src_hbm, dst_hbm, zeros_hbm, out_hbm,
        src_v, dst_v, rows_v, acc_sh, sem):
    c = lax.axis_index("c")
    s = lax.axis_index("s")
    w = c * NS + s
    pltpu.sync_copy(zeros_hbm.at[pl.ds(s * RPS, RPS)],
                    acc_sh.at[pl.ds(s * RPS, RPS)])
    pltpu.sync_copy(src_hbm.at[pl.ds(w * CPW, CPW)], src_v)
    pltpu.sync_copy(dst_hbm.at[pl.ds(w * CPW, CPW)], dst_v)
    plsc.subcore_barrier()

    def body(j, carry):
      pltpu.async_copy(ht_hbm.at[src_v.at[j]], rows_v, sem).wait()
      pltpu.sync_copy(rows_v, acc_sh.at[dst_v.at[j]], add=True)
      return carry

    lax.fori_loop(0, CPW, body, 0)
    plsc.subcore_barrier()
    pltpu.sync_copy(acc_sh.at[pl.ds(s * RPS, RPS)],
                    out_hbm.at[c, pl.ds(s * RPS, RPS)])

  return k


_SC_DEGREE = _make_sc_degree()
_SC_AGG32 = _make_sc_agg(32)
_SC_AGG64 = _make_sc_agg(64)


def _row_mask(ncols):
  rows = lax.broadcasted_iota(jnp.int32, (NPAD, ncols), 0)
  return rows < NNODES


def _bn_relu(a, g, b, mask):
  am = jnp.where(mask, a, 0.0)
  mu = jnp.sum(am, axis=0, keepdims=True) / NNODES
  d = jnp.where(mask, a - mu, 0.0)
  var = jnp.sum(d * d, axis=0, keepdims=True) / NNODES
  h = (a - mu) * lax.rsqrt(var + EPS) * g + b
  return jnp.maximum(h, 0.0)


def _tc_pre_body(deg_ref, x_ref, w1_ref, ht1_ref, dinv_ref):
  deg = deg_ref[0] + deg_ref[1] + 1.0
  dinv = jnp.where(_row_mask(8), lax.rsqrt(deg), 0.0)
  dinv_ref[...] = dinv
  t1 = jnp.dot(x_ref[...], w1_ref[...], preferred_element_type=jnp.float32)
  ht1_ref[...] = dinv[:, 0:1] * t1


_TC_PRE = pl.pallas_call(
    _tc_pre_body,
    out_shape=(jax.ShapeDtypeStruct((NPAD, 32), jnp.float32),
               jax.ShapeDtypeStruct((NPAD, 8), jnp.float32)))


def _tc_mid1_body(acc_ref, ht_ref, dinv_ref, b_ref, g_ref, be_ref, out_ref):
  dinv = dinv_ref[...][:, 0:1]
  a = dinv * (acc_ref[0] + acc_ref[1] + ht_ref[...]) + b_ref[...]
  mask = _row_mask(a.shape[1])
  h = _bn_relu(a, g_ref[...], be_ref[...], mask)
  out_ref[...] = jnp.where(mask, dinv * h, 0.0)


_TC_MID1 = pl.pallas_call(
    _tc_mid1_body,
    out_shape=jax.ShapeDtypeStruct((NPAD, 32), jnp.float32))


def _tc_mid2_body(acc_ref, ht_ref, dinv_ref, w_ref, b_ref, g_ref, be_ref,
                  out_ref):
  dinv = dinv_ref[...][:, 0:1]
  u = dinv * (acc_ref[0] + acc_ref[1] + ht_ref[...])
  a = jnp.dot(u, w_ref[...], preferred_element_type=jnp.float32) + b_ref[...]
  mask = _row_mask(a.shape[1])
  h = _bn_relu(a, g_ref[...], be_ref[...], mask)
  out_ref[...] = jnp.where(mask, dinv * h, 0.0)


_TC_MID2 = pl.pallas_call(
    _tc_mid2_body,
    out_shape=jax.ShapeDtypeStruct((NPAD, 64), jnp.float32))


def _tc_fin_body(acc_ref, ht_ref, dinv_ref, w3_ref, b3_ref, g3_ref, be3_ref,
                 wf_ref, bf_ref, out_ref):
  dinv = dinv_ref[...][:, 0:1]
  u = dinv * (acc_ref[0] + acc_ref[1] + ht_ref[...])
  a = jnp.dot(u, w3_ref[...], preferred_element_type=jnp.float32) + b3_ref[...]
  mask = _row_mask(a.shape[1])
  h = _bn_relu(a, g3_ref[...], be3_ref[...], mask)
  logits = jnp.dot(h, wf_ref[...], preferred_element_type=jnp.float32)
  logits = logits + bf_ref[...]
  m = jnp.max(logits, axis=1, keepdims=True)
  lse = jnp.log(jnp.sum(jnp.exp(logits - m), axis=1, keepdims=True)) + m
  out_ref[...] = logits - lse


_TC_FIN = pl.pallas_call(
    _tc_fin_body,
    out_shape=jax.ShapeDtypeStruct((NPAD, 16), jnp.float32))


def kernel(x, edge_index, W1, b1, g1, be1, W2, b2, g2, be2, W3, b3, g3, be3,
           Wf, bf):
  src = edge_index[0]
  dst = edge_index[1]
  fill = jnp.full((EPAD - NEDGES,), NNODES, jnp.int32)
  src2 = jnp.concatenate([src, fill]).reshape(NW * CPW, CHUNK)
  dst2 = jnp.concatenate([dst, fill]).reshape(NW * CPW, CHUNK)
  xp = jnp.pad(x, ((0, NPAD - NNODES), (0, 0)))
  z8 = jnp.zeros((NPAD, 8), jnp.float32)
  z32 = jnp.zeros((NPAD, 32), jnp.float32)
  z64 = jnp.zeros((NPAD, 64), jnp.float32)
  ones8 = jnp.ones((CHUNK, 8), jnp.float32)
  wfp = jnp.pad(Wf, ((0, 0), (0, 6)))
  bfp = jnp.concatenate([bf, jnp.full((6,), -1e30, jnp.float32)])

  degp = _SC_DEGREE(dst2, ones8, z8)
  ht1, dinv8 = _TC_PRE(degp, xp, W1)
  acc1 = _SC_AGG32(ht1, src2, dst2, z32)
  ht2 = _TC_MID1(acc1, ht1, dinv8, b1.reshape(1, -1), g1.reshape(1, -1),
                 be1.reshape(1, -1))
  acc2 = _SC_AGG32(ht2, src2, dst2, z32)
  ht3 = _TC_MID2(acc2, ht2, dinv8, W2, b2.reshape(1, -1), g2.reshape(1, -1),
                 be2.reshape(1, -1))
  acc3 = _SC_AGG64(ht3, src2, dst2, z64)
  outp = _TC_FIN(acc3, ht3, dinv8, W3, b3.reshape(1, -1), g3.reshape(1, -1),
                 be3.reshape(1, -1), wfp, bfp.reshape(1, -1))
  return outp[:NNODES, :10]


# speedup vs baseline: 14.3623x; 14.3623x over previous
import functools

import jax
import jax.numpy as jnp
from jax import lax
from jax.experimental import pallas as pl
from jax.experimental.pallas import tpu as pltpu
from jax.experimental.pallas import tpu_sc as plsc

NNODES = 10000
NPAD = 10240
NEDGES = 320000
NC = 2
NS = 16
NW = NC * NS
CHUNK = 128
CPW = 80
EPAD = NW * CPW * CHUNK
RPS = NPAD // NS
EPS = 1e-5


@functools.cache
def _make_sc_degree():
  mesh = plsc.VectorSubcoreMesh(core_axis_name="c", subcore_axis_name="s",
                                num_cores=NC, num_subcores=NS)

  @functools.partial(
      pl.kernel,
      out_type=jax.ShapeDtypeStruct((NC, NPAD, 8), jnp.float32),
      mesh=mesh,
      compiler_params=pltpu.CompilerParams(use_tc_tiling_on_sc=False),
      scratch_types=[
          pltpu.VMEM((CPW, CHUNK), jnp.int32),
          pltpu.VMEM((CHUNK, 8), jnp.float32),
          pltpu.VMEM_SHARED((NPAD, 8), jnp.float32),
      ],
  )
  def k(dst_hbm, ones_hbm, zeros_hbm, out_hbm, dst_v, ones_v, acc_sh):
    c = lax.axis_index("c")
    s = lax.axis_index("s")
    w = c * NS + s
    pltpu.sync_copy(zeros_hbm.at[pl.ds(s * RPS, RPS)],
                    acc_sh.at[pl.ds(s * RPS, RPS)])
    pltpu.sync_copy(dst_hbm.at[pl.ds(w * CPW, CPW)], dst_v)
    pltpu.sync_copy(ones_hbm, ones_v)
    plsc.subcore_barrier()

    def body(j, carry):
      pltpu.sync_copy(ones_v, acc_sh.at[dst_v.at[j]], add=True)
      return carry

    lax.fori_loop(0, CPW, body, 0)
    plsc.subcore_barrier()
    pltpu.sync_copy(acc_sh.at[pl.ds(s * RPS, RPS)],
                    out_hbm.at[c, pl.ds(s * RPS, RPS)])

  return k


@functools.cache
def _make_sc_agg(feat):
  mesh = plsc.VectorSubcoreMesh(core_axis_name="c", subcore_axis_name="s",
                                num_cores=NC, num_subcores=NS)

  @functools.partial(
      pl.kernel,
      out_type=jax.ShapeDtypeStruct((NC, NPAD, feat), jnp.float32),
      mesh=mesh,
      compiler_params=pltpu.CompilerParams(use_tc_tiling_on_sc=False),
      scratch_types=[
          pltpu.VMEM((CPW, CHUNK), jnp.int32),
          pltpu.VMEM((CPW, CHUNK), jnp.int32),
          pltpu.VMEM((CHUNK, feat), jnp.float32),
          pltpu.VMEM_SHARED((NPAD, feat), jnp.float32),
          pltpu.SemaphoreType.DMA,
      ],
  )
  def k(ht_hbm, src_hbm, dst_hbm, zeros_hbm, out_hbm,
        src_v, dst_v, rows_v, acc_sh, sem):
    c = lax.axis_index("c")
    s = lax.axis_index("s")
    w = c * NS + s
    pltpu.sync_copy(zeros_hbm.at[pl.ds(s * RPS, RPS)],
                    acc_sh.at[pl.ds(s * RPS, RPS)])
    pltpu.sync_copy(src_hbm.at[pl.ds(w * CPW, CPW)], src_v)
    pltpu.sync_copy(dst_hbm.at[pl.ds(w * CPW, CPW)], dst_v)
    plsc.subcore_barrier()

    def body(j, carry):
      pltpu.async_copy(ht_hbm.at[src_v.at[j]], rows_v, sem).wait()
      pltpu.sync_copy(rows_v, acc_sh.at[dst_v.at[j]], add=True)
      return carry

    lax.fori_loop(0, CPW, body, 0)
    plsc.subcore_barrier()
    pltpu.sync_copy(acc_sh.at[pl.ds(s * RPS, RPS)],
                    out_hbm.at[c, pl.ds(s * RPS, RPS)])

  return k


def _row_mask(ncols):
  rows = lax.broadcasted_iota(jnp.int32, (NPAD, ncols), 0)
  return rows < NNODES


def _bn_relu(a, g, b, mask):
  am = jnp.where(mask, a, 0.0)
  mu = jnp.sum(am, axis=0, keepdims=True) / NNODES
  d = jnp.where(mask, a - mu, 0.0)
  var = jnp.sum(d * d, axis=0, keepdims=True) / NNODES
  h = (a - mu) * lax.rsqrt(var + EPS) * g + b
  return jnp.maximum(h, 0.0)


def _tc_pre_body(deg_ref, x_ref, w1_ref, ht1_ref, dinv_ref):
  deg = deg_ref[0] + deg_ref[1] + 1.0
  dinv = jnp.where(_row_mask(8), lax.rsqrt(deg), 0.0)
  dinv_ref[...] = dinv
  t1 = jnp.dot(x_ref[...], w1_ref[...], preferred_element_type=jnp.float32)
  ht1_ref[...] = dinv[:, 0:1] * t1


_TC_PRE = pl.pallas_call(
    _tc_pre_body,
    out_shape=(jax.ShapeDtypeStruct((NPAD, 32), jnp.float32),
               jax.ShapeDtypeStruct((NPAD, 8), jnp.float32)))


def _tc_mid1_body(acc_ref, ht_ref, dinv_ref, b_ref, g_ref, be_ref, out_ref):
  dinv = dinv_ref[...][:, 0:1]
  a = dinv * (acc_ref[0] + acc_ref[1] + ht_ref[...]) + b_ref[...]
  mask = _row_mask(a.shape[1])
  h = _bn_relu(a, g_ref[...], be_ref[...], mask)
  out_ref[...] = jnp.where(mask, dinv * h, 0.0)


_TC_MID1 = pl.pallas_call(
    _tc_mid1_body,
    out_shape=jax.ShapeDtypeStruct((NPAD, 32), jnp.float32))


def _tc_mid2_body(acc_ref, ht_ref, dinv_ref, w_ref, b_ref, g_ref, be_ref,
                  out_ref):
  dinv = dinv_ref[...][:, 0:1]
  u = dinv * (acc_ref[0] + acc_ref[1] + ht_ref[...])
  a = jnp.dot(u, w_ref[...], preferred_element_type=jnp.float32) + b_ref[...]
  mask = _row_mask(a.shape[1])
  h = _bn_relu(a, g_ref[...], be_ref[...], mask)
  out_ref[...] = jnp.where(mask, dinv * h, 0.0)


_TC_MID2 = pl.pallas_call(
    _tc_mid2_body,
    out_shape=jax.ShapeDtypeStruct((NPAD, 64), jnp.float32))


def _tc_fin_body(acc_ref, ht_ref, dinv_ref, w3_ref, b3_ref, g3_ref, be3_ref,
                 wf_ref, bf_ref, out_ref):
  dinv = dinv_ref[...][:, 0:1]
  u = dinv * (acc_ref[0] + acc_ref[1] + ht_ref[...])
  a = jnp.dot(u, w3_ref[...], preferred_element_type=jnp.float32) + b3_ref[...]
  mask = _row_mask(a.shape[1])
  h = _bn_relu(a, g3_ref[...], be3_ref[...], mask)
  logits = jnp.dot(h, wf_ref[...], preferred_element_type=jnp.float32)
  logits = logits + bf_ref[...]
  m = jnp.max(logits, axis=1, keepdims=True)
  lse = jnp.log(jnp.sum(jnp.exp(logits - m), axis=1, keepdims=True)) + m
  out_ref[...] = logits - lse


_TC_FIN = pl.pallas_call(
    _tc_fin_body,
    out_shape=jax.ShapeDtypeStruct((NPAD, 16), jnp.float32))


def kernel(x, edge_index, W1, b1, g1, be1, W2, b2, g2, be2, W3, b3, g3, be3,
           Wf, bf):
  src = edge_index[0]
  dst = edge_index[1]
  fill = jnp.full((EPAD - NEDGES,), NNODES, jnp.int32)
  src2 = jnp.concatenate([src, fill]).reshape(NW * CPW, CHUNK)
  dst2 = jnp.concatenate([dst, fill]).reshape(NW * CPW, CHUNK)
  xp = jnp.pad(x, ((0, NPAD - NNODES), (0, 0)))
  z8 = jnp.zeros((NPAD, 8), jnp.float32)
  z32 = jnp.zeros((NPAD, 32), jnp.float32)
  z64 = jnp.zeros((NPAD, 64), jnp.float32)
  ones8 = jnp.ones((CHUNK, 8), jnp.float32)
  wfp = jnp.pad(Wf, ((0, 0), (0, 6)))
  bfp = jnp.concatenate([bf, jnp.full((6,), -1e30, jnp.float32)])

  degp = _make_sc_degree()(dst2, ones8, z8)
  ht1, dinv8 = _TC_PRE(degp, xp, W1)
  acc1 = _make_sc_agg(32)(ht1, src2, dst2, z32)
  ht2 = _TC_MID1(acc1, ht1, dinv8, b1.reshape(1, -1), g1.reshape(1, -1),
                 be1.reshape(1, -1))
  acc2 = _make_sc_agg(32)(ht2, src2, dst2, z32)
  ht3 = _TC_MID2(acc2, ht2, dinv8, W2, b2.reshape(1, -1), g2.reshape(1, -1),
                 be2.reshape(1, -1))
  acc3 = _make_sc_agg(64)(ht3, src2, dst2, z64)
  outp = _TC_FIN(acc3, ht3, dinv8, W3, b3.reshape(1, -1), g3.reshape(1, -1),
                 be3.reshape(1, -1), wfp, bfp.reshape(1, -1))
  return outp[:NNODES, :10]


# speedup vs baseline: 17.0162x; 1.1848x over previous
import functools

import jax
import jax.numpy as jnp
from jax import lax
from jax.experimental import pallas as pl
from jax.experimental.pallas import tpu as pltpu
from jax.experimental.pallas import tpu_sc as plsc

NNODES = 10000
NPAD = 10240
NEDGES = 320000
NC = 2
NS = 16
NW = NC * NS
CHUNK = 128
CPW = 80
EPAD = NW * CPW * CHUNK
RPS = NPAD // NS
EPS = 1e-5


RING = 8
NBURST = CPW // RING


@functools.cache
def _make_sc_degree():
  mesh = plsc.VectorSubcoreMesh(core_axis_name="c", subcore_axis_name="s",
                                num_cores=NC, num_subcores=NS)

  @functools.partial(
      pl.kernel,
      out_type=jax.ShapeDtypeStruct((NC, NPAD, 8), jnp.float32),
      mesh=mesh,
      compiler_params=pltpu.CompilerParams(use_tc_tiling_on_sc=False),
      scratch_types=[
          pltpu.VMEM((CPW, CHUNK), jnp.int32),
          pltpu.VMEM((CHUNK, 8), jnp.float32),
          pltpu.VMEM_SHARED((NPAD, 8), jnp.float32),
      ] + [pltpu.SemaphoreType.DMA] * RING,
  )
  def k(dst_hbm, ones_hbm, zeros_hbm, out_hbm, dst_v, ones_v, acc_sh, *sems):
    c = lax.axis_index("c")
    s = lax.axis_index("s")
    w = c * NS + s
    pltpu.sync_copy(zeros_hbm.at[pl.ds(s * RPS, RPS)],
                    acc_sh.at[pl.ds(s * RPS, RPS)])
    pltpu.sync_copy(dst_hbm.at[pl.ds(w * CPW, CPW)], dst_v)
    pltpu.sync_copy(ones_hbm, ones_v)
    plsc.subcore_barrier()

    def scat(j, kslot):
      return pltpu.make_async_copy(ones_v, acc_sh.at[dst_v.at[j]],
                                   sems[kslot])

    for kk in range(RING):
      scat(kk, kk).start(add=True)

    def body(t, carry):
      base = t * RING
      for kk in range(RING):
        scat(base + kk, kk).wait()
        scat(base + kk, kk).start(add=True)
      return carry

    lax.fori_loop(1, NBURST, body, 0)
    for kk in range(RING):
      scat(kk, kk).wait()
    plsc.subcore_barrier()
    pltpu.sync_copy(acc_sh.at[pl.ds(s * RPS, RPS)],
                    out_hbm.at[c, pl.ds(s * RPS, RPS)])

  return k


@functools.cache
def _make_sc_agg(feat):
  mesh = plsc.VectorSubcoreMesh(core_axis_name="c", subcore_axis_name="s",
                                num_cores=NC, num_subcores=NS)

  @functools.partial(
      pl.kernel,
      out_type=jax.ShapeDtypeStruct((NC, NPAD, feat), jnp.float32),
      mesh=mesh,
      compiler_params=pltpu.CompilerParams(use_tc_tiling_on_sc=False),
      scratch_types=[
          pltpu.VMEM((CPW, CHUNK), jnp.int32),
          pltpu.VMEM((CPW, CHUNK), jnp.int32),
          pltpu.VMEM_SHARED((NPAD, feat), jnp.float32),
      ] + [pltpu.VMEM((CHUNK, feat), jnp.float32)] * RING
        + [pltpu.SemaphoreType.DMA] * (2 * RING),
  )
  def k(ht_hbm, src_hbm, dst_hbm, zeros_hbm, out_hbm,
        src_v, dst_v, acc_sh, *rest):
    rows = rest[:RING]
    gsem = rest[RING:2 * RING]
    ssem = rest[2 * RING:]
    c = lax.axis_index("c")
    s = lax.axis_index("s")
    w = c * NS + s
    pltpu.sync_copy(zeros_hbm.at[pl.ds(s * RPS, RPS)],
                    acc_sh.at[pl.ds(s * RPS, RPS)])
    pltpu.sync_copy(src_hbm.at[pl.ds(w * CPW, CPW)], src_v)
    pltpu.sync_copy(dst_hbm.at[pl.ds(w * CPW, CPW)], dst_v)
    plsc.subcore_barrier()

    def gat(j, kslot):
      return pltpu.make_async_copy(ht_hbm.at[src_v.at[j]], rows[kslot],
                                   gsem[kslot])

    def scat(j, kslot):
      return pltpu.make_async_copy(rows[kslot], acc_sh.at[dst_v.at[j]],
                                   ssem[kslot])

    for kk in range(RING):
      gat(kk, kk).start()
    for kk in range(RING):
      gat(kk, kk).wait()
      scat(kk, kk).start(add=True)

    def body(t, carry):
      base = t * RING
      for kk in range(RING):
        scat(base + kk, kk).wait()
        gat(base + kk, kk).start()
      for kk in range(RING):
        gat(base + kk, kk).wait()
        scat(base + kk, kk).start(add=True)
      return carry

    lax.fori_loop(1, NBURST, body, 0)
    for kk in range(RING):
      scat(kk, kk).wait()
    plsc.subcore_barrier()
    pltpu.sync_copy(acc_sh.at[pl.ds(s * RPS, RPS)],
                    out_hbm.at[c, pl.ds(s * RPS, RPS)])

  return k


def _row_mask(ncols):
  rows = lax.broadcasted_iota(jnp.int32, (NPAD, ncols), 0)
  return rows < NNODES


def _bn_relu(a, g, b, mask):
  am = jnp.where(mask, a, 0.0)
  mu = jnp.sum(am, axis=0, keepdims=True) / NNODES
  d = jnp.where(mask, a - mu, 0.0)
  var = jnp.sum(d * d, axis=0, keepdims=True) / NNODES
  h = (a - mu) * lax.rsqrt(var + EPS) * g + b
  return jnp.maximum(h, 0.0)


def _tc_pre_body(deg_ref, x_ref, w1_ref, ht1_ref, dinv_ref):
  deg = deg_ref[0] + deg_ref[1] + 1.0
  dinv = jnp.where(_row_mask(8), lax.rsqrt(deg), 0.0)
  dinv_ref[...] = dinv
  t1 = jnp.dot(x_ref[...], w1_ref[...], preferred_element_type=jnp.float32)
  ht1_ref[...] = dinv[:, 0:1] * t1


_TC_PRE = pl.pallas_call(
    _tc_pre_body,
    out_shape=(jax.ShapeDtypeStruct((NPAD, 32), jnp.float32),
               jax.ShapeDtypeStruct((NPAD, 8), jnp.float32)))


def _tc_mid1_body(acc_ref, ht_ref, dinv_ref, b_ref, g_ref, be_ref, out_ref):
  dinv = dinv_ref[...][:, 0:1]
  a = dinv * (acc_ref[0] + acc_ref[1] + ht_ref[...]) + b_ref[...]
  mask = _row_mask(a.shape[1])
  h = _bn_relu(a, g_ref[...], be_ref[...], mask)
  out_ref[...] = jnp.where(mask, dinv * h, 0.0)


_TC_MID1 = pl.pallas_call(
    _tc_mid1_body,
    out_shape=jax.ShapeDtypeStruct((NPAD, 32), jnp.float32))


def _tc_mid2_body(acc_ref, ht_ref, dinv_ref, w_ref, b_ref, g_ref, be_ref,
                  out_ref):
  dinv = dinv_ref[...][:, 0:1]
  u = dinv * (acc_ref[0] + acc_ref[1] + ht_ref[...])
  a = jnp.dot(u, w_ref[...], preferred_element_type=jnp.float32) + b_ref[...]
  mask = _row_mask(a.shape[1])
  h = _bn_relu(a, g_ref[...], be_ref[...], mask)
  out_ref[...] = jnp.where(mask, dinv * h, 0.0)


_TC_MID2 = pl.pallas_call(
    _tc_mid2_body,
    out_shape=jax.ShapeDtypeStruct((NPAD, 64), jnp.float32))


def _tc_fin_body(acc_ref, ht_ref, dinv_ref, w3_ref, b3_ref, g3_ref, be3_ref,
                 wf_ref, bf_ref, out_ref):
  dinv = dinv_ref[...][:, 0:1]
  u = dinv * (acc_ref[0] + acc_ref[1] + ht_ref[...])
  a = jnp.dot(u, w3_ref[...], preferred_element_type=jnp.float32) + b3_ref[...]
  mask = _row_mask(a.shape[1])
  h = _bn_relu(a, g3_ref[...], be3_ref[...], mask)
  logits = jnp.dot(h, wf_ref[...], preferred_element_type=jnp.float32)
  logits = logits + bf_ref[...]
  m = jnp.max(logits, axis=1, keepdims=True)
  lse = jnp.log(jnp.sum(jnp.exp(logits - m), axis=1, keepdims=True)) + m
  out_ref[...] = logits - lse


_TC_FIN = pl.pallas_call(
    _tc_fin_body,
    out_shape=jax.ShapeDtypeStruct((NPAD, 16), jnp.float32))


def kernel(x, edge_index, W1, b1, g1, be1, W2, b2, g2, be2, W3, b3, g3, be3,
           Wf, bf):
  src = edge_index[0]
  dst = edge_index[1]
  fill = jnp.full((EPAD - NEDGES,), NNODES, jnp.int32)
  src2 = jnp.concatenate([src, fill]).reshape(NW * CPW, CHUNK)
  dst2 = jnp.concatenate([dst, fill]).reshape(NW * CPW, CHUNK)
  xp = jnp.pad(x, ((0, NPAD - NNODES), (0, 0)))
  z8 = jnp.zeros((NPAD, 8), jnp.float32)
  z32 = jnp.zeros((NPAD, 32), jnp.float32)
  z64 = jnp.zeros((NPAD, 64), jnp.float32)
  ones8 = jnp.ones((CHUNK, 8), jnp.float32)
  wfp = jnp.pad(Wf, ((0, 0), (0, 6)))
  bfp = jnp.concatenate([bf, jnp.full((6,), -1e30, jnp.float32)])

  degp = _make_sc_degree()(dst2, ones8, z8)
  ht1, dinv8 = _TC_PRE(degp, xp, W1)
  acc1 = _make_sc_agg(32)(ht1, src2, dst2, z32)
  ht2 = _TC_MID1(acc1, ht1, dinv8, b1.reshape(1, -1), g1.reshape(1, -1),
                 be1.reshape(1, -1))
  acc2 = _make_sc_agg(32)(ht2, src2, dst2, z32)
  ht3 = _TC_MID2(acc2, ht2, dinv8, W2, b2.reshape(1, -1), g2.reshape(1, -1),
                 be2.reshape(1, -1))
  acc3 = _make_sc_agg(64)(ht3, src2, dst2, z64)
  outp = _TC_FIN(acc3, ht3, dinv8, W3, b3.reshape(1, -1), g3.reshape(1, -1),
                 be3.reshape(1, -1), wfp, bfp.reshape(1, -1))
  return outp[:NNODES, :10]


# speedup vs baseline: 17.2041x; 1.0110x over previous
import functools

import jax
import jax.numpy as jnp
from jax import lax
from jax.experimental import pallas as pl
from jax.experimental.pallas import tpu as pltpu
from jax.experimental.pallas import tpu_sc as plsc

NNODES = 10000
NPAD = 10240
NEDGES = 320000
NC = 2
NS = 16
NW = NC * NS
CHUNK = 128
CPW = 80
EPAD = NW * CPW * CHUNK
RPS = NPAD // NS
EPS = 1e-5


RING = 8
NBURST = CPW // RING


@functools.cache
def _make_sc_degree():
  mesh = plsc.VectorSubcoreMesh(core_axis_name="c", subcore_axis_name="s",
                                num_cores=NC, num_subcores=NS)

  @functools.partial(
      pl.kernel,
      out_type=jax.ShapeDtypeStruct((NC, NPAD, 8), jnp.float32),
      mesh=mesh,
      compiler_params=pltpu.CompilerParams(use_tc_tiling_on_sc=False),
      scratch_types=[
          pltpu.VMEM((CPW, CHUNK), jnp.int32),
          pltpu.VMEM((CHUNK, 8), jnp.float32),
          pltpu.VMEM_SHARED((NPAD, 8), jnp.float32),
      ] + [pltpu.SemaphoreType.DMA] * RING,
  )
  def k(dst_hbm, ones_hbm, zeros_hbm, out_hbm, dst_v, ones_v, acc_sh, *sems):
    c = lax.axis_index("c")
    s = lax.axis_index("s")
    w = c * NS + s
    pltpu.sync_copy(zeros_hbm.at[pl.ds(s * RPS, RPS)],
                    acc_sh.at[pl.ds(s * RPS, RPS)])
    pltpu.sync_copy(dst_hbm.at[pl.ds(w * CPW, CPW)], dst_v)
    pltpu.sync_copy(ones_hbm, ones_v)
    plsc.subcore_barrier()

    def scat(j, kslot):
      return pltpu.make_async_copy(ones_v, acc_sh.at[dst_v.at[j]],
                                   sems[kslot])

    for kk in range(RING):
      scat(kk, kk).start(add=True)

    def body(t, carry):
      base = t * RING
      for kk in range(RING):
        scat(base + kk, kk).wait()
        scat(base + kk, kk).start(add=True)
      return carry

    lax.fori_loop(1, NBURST, body, 0)
    for kk in range(RING):
      scat(kk, kk).wait()
    plsc.subcore_barrier()
    pltpu.sync_copy(acc_sh.at[pl.ds(s * RPS, RPS)],
                    out_hbm.at[c, pl.ds(s * RPS, RPS)])

  return k


@functools.cache
def _make_sc_agg(feat):
  mesh = plsc.VectorSubcoreMesh(core_axis_name="c", subcore_axis_name="s",
                                num_cores=NC, num_subcores=NS)

  @functools.partial(
      pl.kernel,
      out_type=jax.ShapeDtypeStruct((NC, NPAD, feat), jnp.float32),
      mesh=mesh,
      compiler_params=pltpu.CompilerParams(use_tc_tiling_on_sc=False),
      scratch_types=[
          pltpu.VMEM((CPW, CHUNK), jnp.int32),
          pltpu.VMEM((CPW, CHUNK), jnp.int32),
          pltpu.VMEM_SHARED((NPAD, feat), jnp.float32),
      ] + [pltpu.VMEM((CHUNK, feat), jnp.float32)] * RING
        + [pltpu.SemaphoreType.DMA] * (2 * RING),
  )
  def k(ht_hbm, src_hbm, dst_hbm, zeros_hbm, out_hbm,
        src_v, dst_v, acc_sh, *rest):
    rows = rest[:RING]
    gsem = rest[RING:2 * RING]
    ssem = rest[2 * RING:]
    c = lax.axis_index("c")
    s = lax.axis_index("s")
    w = c * NS + s
    pltpu.sync_copy(zeros_hbm.at[pl.ds(s * RPS, RPS)],
                    acc_sh.at[pl.ds(s * RPS, RPS)])
    pltpu.sync_copy(src_hbm.at[pl.ds(w * CPW, CPW)], src_v)
    pltpu.sync_copy(dst_hbm.at[pl.ds(w * CPW, CPW)], dst_v)
    plsc.subcore_barrier()

    def gat(j, kslot):
      return pltpu.make_async_copy(ht_hbm.at[src_v.at[j]], rows[kslot],
                                   gsem[kslot])

    def scat(j, kslot):
      return pltpu.make_async_copy(rows[kslot], acc_sh.at[dst_v.at[j]],
                                   ssem[kslot])

    for kk in range(RING):
      gat(kk, kk).start()
    for kk in range(RING):
      gat(kk, kk).wait()
      scat(kk, kk).start(add=True)

    def body(t, carry):
      base = t * RING
      for kk in range(RING):
        scat(base + kk, kk).wait()
        gat(base + kk, kk).start()
      for kk in range(RING):
        gat(base + kk, kk).wait()
        scat(base + kk, kk).start(add=True)
      return carry

    lax.fori_loop(1, NBURST, body, 0)
    for kk in range(RING):
      scat(kk, kk).wait()
    plsc.subcore_barrier()
    pltpu.sync_copy(acc_sh.at[pl.ds(s * RPS, RPS)],
                    out_hbm.at[c, pl.ds(s * RPS, RPS)])

  return k


def _row_mask(ncols):
  rows = lax.broadcasted_iota(jnp.int32, (NPAD, ncols), 0)
  return rows < NNODES


def _bn_relu(a, g, b, mask):
  am = jnp.where(mask, a, 0.0)
  mu = jnp.sum(am, axis=0, keepdims=True) / NNODES
  d = jnp.where(mask, a - mu, 0.0)
  var = jnp.sum(d * d, axis=0, keepdims=True) / NNODES
  h = (a - mu) * lax.rsqrt(var + EPS) * g + b
  return jnp.maximum(h, 0.0)


def _tc_pre_body(deg_ref, x_ref, w1_ref, ht1_ref, dinv_ref):
  deg = deg_ref[0] + deg_ref[1] + 1.0
  dinv = jnp.where(_row_mask(8), lax.rsqrt(deg), 0.0)
  dinv_ref[...] = dinv
  t1 = jnp.dot(x_ref[...], w1_ref[...], preferred_element_type=jnp.float32)
  ht1_ref[...] = dinv[:, 0:1] * t1


_TC_PRE = pl.pallas_call(
    _tc_pre_body,
    out_shape=(jax.ShapeDtypeStruct((NPAD, 32), jnp.float32),
               jax.ShapeDtypeStruct((NPAD, 8), jnp.float32)))


def _tc_mid1_body(acc_ref, ht_ref, dinv_ref, b_ref, g_ref, be_ref, out_ref):
  dinv = dinv_ref[...][:, 0:1]
  a = dinv * (acc_ref[0] + acc_ref[1] + ht_ref[...]) + b_ref[...]
  mask = _row_mask(a.shape[1])
  h = _bn_relu(a, g_ref[...], be_ref[...], mask)
  out_ref[...] = jnp.where(mask, dinv * h, 0.0)


_TC_MID1 = pl.pallas_call(
    _tc_mid1_body,
    out_shape=jax.ShapeDtypeStruct((NPAD, 32), jnp.float32))


def _tc_mid2_body(acc_ref, ht_ref, dinv_ref, w_ref, b_ref, g_ref, be_ref,
                  out_ref):
  dinv = dinv_ref[...][:, 0:1]
  u = dinv * (acc_ref[0] + acc_ref[1] + ht_ref[...])
  a = jnp.dot(u, w_ref[...], preferred_element_type=jnp.float32) + b_ref[...]
  mask = _row_mask(a.shape[1])
  h = _bn_relu(a, g_ref[...], be_ref[...], mask)
  out_ref[...] = jnp.where(mask, dinv * h, 0.0)


_TC_MID2 = pl.pallas_call(
    _tc_mid2_body,
    out_shape=jax.ShapeDtypeStruct((NPAD, 64), jnp.float32))


def _tc_fin_body(acc_ref, ht_ref, dinv_ref, w3_ref, b3_ref, g3_ref, be3_ref,
                 wf_ref, bf_ref, out_ref):
  dinv = dinv_ref[...][:, 0:1]
  u = dinv * (acc_ref[0] + acc_ref[1] + ht_ref[...])
  a = jnp.dot(u, w3_ref[...], preferred_element_type=jnp.float32) + b3_ref[...]
  mask = _row_mask(a.shape[1])
  h = _bn_relu(a, g3_ref[...], be3_ref[...], mask)
  logits = jnp.dot(h, wf_ref[...], preferred_element_type=jnp.float32)
  logits = logits + bf_ref[...]
  m = jnp.max(logits, axis=1, keepdims=True)
  lse = jnp.log(jnp.sum(jnp.exp(logits - m), axis=1, keepdims=True)) + m
  out_ref[...] = logits - lse


_TC_FIN = pl.pallas_call(
    _tc_fin_body,
    out_shape=jax.ShapeDtypeStruct((NPAD, 16), jnp.float32))


def kernel(x, edge_index, W1, b1, g1, be1, W2, b2, g2, be2, W3, b3, g3, be3,
           Wf, bf):
  src = edge_index[0]
  dst = edge_index[1]
  fill_src = jnp.full((EPAD - NEDGES,), NNODES, jnp.int32)
  fill_dst = NNODES + jnp.arange(EPAD - NEDGES, dtype=jnp.int32) % (
      NPAD - NNODES)
  src2 = jnp.concatenate([src, fill_src]).reshape(NW * CPW, CHUNK)
  dst2 = jnp.concatenate([dst, fill_dst]).reshape(NW * CPW, CHUNK)
  xp = jnp.pad(x, ((0, NPAD - NNODES), (0, 0)))
  z8 = jnp.zeros((NPAD, 8), jnp.float32)
  z32 = jnp.zeros((NPAD, 32), jnp.float32)
  z64 = jnp.zeros((NPAD, 64), jnp.float32)
  ones8 = jnp.ones((CHUNK, 8), jnp.float32)
  wfp = jnp.pad(Wf, ((0, 0), (0, 6)))
  bfp = jnp.concatenate([bf, jnp.full((6,), -1e30, jnp.float32)])

  degp = _make_sc_degree()(dst2, ones8, z8)
  ht1, dinv8 = _TC_PRE(degp, xp, W1)
  acc1 = _make_sc_agg(32)(ht1, src2, dst2, z32)
  ht2 = _TC_MID1(acc1, ht1, dinv8, b1.reshape(1, -1), g1.reshape(1, -1),
                 be1.reshape(1, -1))
  acc2 = _make_sc_agg(32)(ht2, src2, dst2, z32)
  ht3 = _TC_MID2(acc2, ht2, dinv8, W2, b2.reshape(1, -1), g2.reshape(1, -1),
                 be2.reshape(1, -1))
  acc3 = _make_sc_agg(64)(ht3, src2, dst2, z64)
  outp = _TC_FIN(acc3, ht3, dinv8, W3, b3.reshape(1, -1), g3.reshape(1, -1),
                 be3.reshape(1, -1), wfp, bfp.reshape(1, -1))
  return outp[:NNODES, :10]


# speedup vs baseline: 22.2352x; 1.2924x over previous
import functools

import jax
import jax.numpy as jnp
from jax import lax
from jax.experimental import pallas as pl
from jax.experimental.pallas import tpu as pltpu
from jax.experimental.pallas import tpu_sc as plsc

NNODES = 10000
NPAD = 10240
NEDGES = 320000
NC = 2
NS = 16
NW = NC * NS
CHUNK = 128
CPW = 80
EPAD = NW * CPW * CHUNK
RPS = NPAD // NS
EPS = 1e-5


RING = 8
NBURST = CPW // RING


@functools.cache
def _make_sc_degree():
  mesh = plsc.VectorSubcoreMesh(core_axis_name="c", subcore_axis_name="s",
                                num_cores=NC, num_subcores=NS)

  @functools.partial(
      pl.kernel,
      out_type=jax.ShapeDtypeStruct((NC, NPAD, 8), jnp.float32),
      mesh=mesh,
      compiler_params=pltpu.CompilerParams(use_tc_tiling_on_sc=False),
      scratch_types=[
          pltpu.VMEM((CPW, CHUNK), jnp.int32),
          pltpu.VMEM((CHUNK, 8), jnp.float32),
          pltpu.VMEM_SHARED((NPAD, 8), jnp.float32),
      ] + [pltpu.SemaphoreType.DMA] * RING,
  )
  def k(dst_hbm, ones_hbm, zeros_hbm, out_hbm, dst_v, ones_v, acc_sh, *sems):
    c = lax.axis_index("c")
    s = lax.axis_index("s")
    w = c * NS + s
    pltpu.sync_copy(zeros_hbm.at[pl.ds(s * RPS, RPS)],
                    acc_sh.at[pl.ds(s * RPS, RPS)])
    pltpu.sync_copy(dst_hbm.at[pl.ds(w * CPW, CPW)], dst_v)
    pltpu.sync_copy(ones_hbm, ones_v)
    plsc.subcore_barrier()

    def scat(j, kslot):
      return pltpu.make_async_copy(ones_v, acc_sh.at[dst_v.at[j]],
                                   sems[kslot])

    for kk in range(RING):
      scat(kk, kk).start(add=True)

    def body(t, carry):
      base = t * RING
      for kk in range(RING):
        scat(base + kk, kk).wait()
        scat(base + kk, kk).start(add=True)
      return carry

    lax.fori_loop(1, NBURST, body, 0)
    for kk in range(RING):
      scat(kk, kk).wait()
    plsc.subcore_barrier()
    pltpu.sync_copy(acc_sh.at[pl.ds(s * RPS, RPS)],
                    out_hbm.at[c, pl.ds(s * RPS, RPS)])

  return k


@functools.cache
def _make_sc_agg(feat, stage):
  mesh = plsc.VectorSubcoreMesh(core_axis_name="c", subcore_axis_name="s",
                                num_cores=NC, num_subcores=NS)

  @functools.partial(
      pl.kernel,
      out_type=jax.ShapeDtypeStruct((NC, NPAD, feat), jnp.float32),
      mesh=mesh,
      compiler_params=pltpu.CompilerParams(use_tc_tiling_on_sc=False),
      scratch_types=[
          pltpu.VMEM((CPW, CHUNK), jnp.int32),
          pltpu.VMEM((CPW, CHUNK), jnp.int32),
          pltpu.VMEM_SHARED((NPAD, feat), jnp.float32),
      ] + ([pltpu.VMEM_SHARED((NPAD, feat), jnp.float32)] if stage else [])
        + [pltpu.VMEM((CHUNK, feat), jnp.float32)] * RING
        + [pltpu.SemaphoreType.DMA] * (2 * RING),
  )
  def k(ht_hbm, src_hbm, dst_hbm, zeros_hbm, out_hbm,
        src_v, dst_v, acc_sh, *rest):
    if stage:
      ht_sh, *rest = rest
    rows = rest[:RING]
    gsem = rest[RING:2 * RING]
    ssem = rest[2 * RING:]
    c = lax.axis_index("c")
    s = lax.axis_index("s")
    w = c * NS + s
    if stage:
      pltpu.sync_copy(ht_hbm.at[pl.ds(s * RPS, RPS)],
                      ht_sh.at[pl.ds(s * RPS, RPS)])
      gsrc = ht_sh
    else:
      gsrc = ht_hbm
    pltpu.sync_copy(zeros_hbm.at[pl.ds(s * RPS, RPS)],
                    acc_sh.at[pl.ds(s * RPS, RPS)])
    pltpu.sync_copy(src_hbm.at[pl.ds(w * CPW, CPW)], src_v)
    pltpu.sync_copy(dst_hbm.at[pl.ds(w * CPW, CPW)], dst_v)
    plsc.subcore_barrier()

    def gat(j, kslot):
      return pltpu.make_async_copy(gsrc.at[src_v.at[j]], rows[kslot],
                                   gsem[kslot])

    def scat(j, kslot):
      return pltpu.make_async_copy(rows[kslot], acc_sh.at[dst_v.at[j]],
                                   ssem[kslot])

    for kk in range(RING):
      gat(kk, kk).start()
    for kk in range(RING):
      gat(kk, kk).wait()
      scat(kk, kk).start(add=True)

    def body(t, carry):
      base = t * RING
      for kk in range(RING):
        scat(base + kk, kk).wait()
        gat(base + kk, kk).start()
      for kk in range(RING):
        gat(base + kk, kk).wait()
        scat(base + kk, kk).start(add=True)
      return carry

    lax.fori_loop(1, NBURST, body, 0)
    for kk in range(RING):
      scat(kk, kk).wait()
    plsc.subcore_barrier()
    pltpu.sync_copy(acc_sh.at[pl.ds(s * RPS, RPS)],
                    out_hbm.at[c, pl.ds(s * RPS, RPS)])

  return k


def _row_mask(ncols):
  rows = lax.broadcasted_iota(jnp.int32, (NPAD, ncols), 0)
  return rows < NNODES


def _bn_relu(a, g, b, mask):
  am = jnp.where(mask, a, 0.0)
  mu = jnp.sum(am, axis=0, keepdims=True) / NNODES
  d = jnp.where(mask, a - mu, 0.0)
  var = jnp.sum(d * d, axis=0, keepdims=True) / NNODES
  h = (a - mu) * lax.rsqrt(var + EPS) * g + b
  return jnp.maximum(h, 0.0)


def _tc_pre_body(deg_ref, x_ref, w1_ref, ht1_ref, dinv_ref):
  deg = deg_ref[0] + deg_ref[1] + 1.0
  dinv = jnp.where(_row_mask(8), lax.rsqrt(deg), 0.0)
  dinv_ref[...] = dinv
  t1 = jnp.dot(x_ref[...], w1_ref[...], preferred_element_type=jnp.float32)
  ht1_ref[...] = dinv[:, 0:1] * t1


_TC_PRE = pl.pallas_call(
    _tc_pre_body,
    out_shape=(jax.ShapeDtypeStruct((NPAD, 32), jnp.float32),
               jax.ShapeDtypeStruct((NPAD, 8), jnp.float32)))


def _tc_mid1_body(acc_ref, ht_ref, dinv_ref, b_ref, g_ref, be_ref, out_ref):
  dinv = dinv_ref[...][:, 0:1]
  a = dinv * (acc_ref[0] + acc_ref[1] + ht_ref[...]) + b_ref[...]
  mask = _row_mask(a.shape[1])
  h = _bn_relu(a, g_ref[...], be_ref[...], mask)
  out_ref[...] = jnp.where(mask, dinv * h, 0.0)


_TC_MID1 = pl.pallas_call(
    _tc_mid1_body,
    out_shape=jax.ShapeDtypeStruct((NPAD, 32), jnp.float32))


def _tc_mid2_body(acc_ref, ht_ref, dinv_ref, w_ref, b_ref, g_ref, be_ref,
                  out_ref):
  dinv = dinv_ref[...][:, 0:1]
  u = dinv * (acc_ref[0] + acc_ref[1] + ht_ref[...])
  a = jnp.dot(u, w_ref[...], preferred_element_type=jnp.float32) + b_ref[...]
  mask = _row_mask(a.shape[1])
  h = _bn_relu(a, g_ref[...], be_ref[...], mask)
  out_ref[...] = jnp.where(mask, dinv * h, 0.0)


_TC_MID2 = pl.pallas_call(
    _tc_mid2_body,
    out_shape=jax.ShapeDtypeStruct((NPAD, 64), jnp.float32))


def _tc_fin_body(acc_ref, ht_ref, dinv_ref, w3_ref, b3_ref, g3_ref, be3_ref,
                 wf_ref, bf_ref, out_ref):
  dinv = dinv_ref[...][:, 0:1]
  u = dinv * (acc_ref[0] + acc_ref[1] + ht_ref[...])
  a = jnp.dot(u, w3_ref[...], preferred_element_type=jnp.float32) + b3_ref[...]
  mask = _row_mask(a.shape[1])
  h = _bn_relu(a, g3_ref[...], be3_ref[...], mask)
  logits = jnp.dot(h, wf_ref[...], preferred_element_type=jnp.float32)
  logits = logits + bf_ref[...]
  m = jnp.max(logits, axis=1, keepdims=True)
  lse = jnp.log(jnp.sum(jnp.exp(logits - m), axis=1, keepdims=True)) + m
  out_ref[...] = logits - lse


_TC_FIN = pl.pallas_call(
    _tc_fin_body,
    out_shape=jax.ShapeDtypeStruct((NPAD, 16), jnp.float32))


def kernel(x, edge_index, W1, b1, g1, be1, W2, b2, g2, be2, W3, b3, g3, be3,
           Wf, bf):
  src = edge_index[0]
  dst = edge_index[1]
  fill_src = jnp.full((EPAD - NEDGES,), NNODES, jnp.int32)
  fill_dst = NNODES + jnp.arange(EPAD - NEDGES, dtype=jnp.int32) % (
      NPAD - NNODES)
  src2 = jnp.concatenate([src, fill_src]).reshape(NW * CPW, CHUNK)
  dst2 = jnp.concatenate([dst, fill_dst]).reshape(NW * CPW, CHUNK)
  xp = jnp.pad(x, ((0, NPAD - NNODES), (0, 0)))
  z8 = jnp.zeros((NPAD, 8), jnp.float32)
  z32 = jnp.zeros((NPAD, 32), jnp.float32)
  z64 = jnp.zeros((NPAD, 64), jnp.float32)
  ones8 = jnp.ones((CHUNK, 8), jnp.float32)
  wfp = jnp.pad(Wf, ((0, 0), (0, 6)))
  bfp = jnp.concatenate([bf, jnp.full((6,), -1e30, jnp.float32)])

  degp = _make_sc_degree()(dst2, ones8, z8)
  ht1, dinv8 = _TC_PRE(degp, xp, W1)
  acc1 = _make_sc_agg(32, True)(ht1, src2, dst2, z32)
  ht2 = _TC_MID1(acc1, ht1, dinv8, b1.reshape(1, -1), g1.reshape(1, -1),
                 be1.reshape(1, -1))
  acc2 = _make_sc_agg(32, True)(ht2, src2, dst2, z32)
  ht3 = _TC_MID2(acc2, ht2, dinv8, W2, b2.reshape(1, -1), g2.reshape(1, -1),
                 be2.reshape(1, -1))
  acc3 = _make_sc_agg(64, False)(ht3, src2, dst2, z64)
  outp = _TC_FIN(acc3, ht3, dinv8, W3, b3.reshape(1, -1), g3.reshape(1, -1),
                 be3.reshape(1, -1), wfp, bfp.reshape(1, -1))
  return outp[:NNODES, :10]


# speedup vs baseline: 33.6547x; 1.5136x over previous
import functools

import jax
import jax.numpy as jnp
from jax import lax
from jax.experimental import pallas as pl
from jax.experimental.pallas import tpu as pltpu
from jax.experimental.pallas import tpu_sc as plsc

NNODES = 10000
NPAD = 10240
NEDGES = 320000
NC = 2
NS = 16
NW = NC * NS
CHUNK = 128
CPW = 80
EPAD = NW * CPW * CHUNK
RPS = NPAD // NS
EPS = 1e-5


RING = 8
NBURST = CPW // RING


@functools.cache
def _make_sc_degree():
  mesh = plsc.VectorSubcoreMesh(core_axis_name="c", subcore_axis_name="s",
                                num_cores=NC, num_subcores=NS)

  @functools.partial(
      pl.kernel,
      out_type=jax.ShapeDtypeStruct((NC, NPAD, 8), jnp.float32),
      mesh=mesh,
      compiler_params=pltpu.CompilerParams(use_tc_tiling_on_sc=False),
      scratch_types=[
          pltpu.VMEM((CPW, CHUNK), jnp.int32),
          pltpu.VMEM((CHUNK, 8), jnp.float32),
          pltpu.VMEM_SHARED((NPAD, 8), jnp.float32),
      ] + [pltpu.SemaphoreType.DMA] * RING,
  )
  def k(dst_hbm, ones_hbm, zeros_hbm, out_hbm, dst_v, ones_v, acc_sh, *sems):
    c = lax.axis_index("c")
    s = lax.axis_index("s")
    w = c * NS + s
    pltpu.sync_copy(zeros_hbm.at[pl.ds(s * RPS, RPS)],
                    acc_sh.at[pl.ds(s * RPS, RPS)])
    pltpu.sync_copy(dst_hbm.at[pl.ds(w * CPW, CPW)], dst_v)
    pltpu.sync_copy(ones_hbm, ones_v)
    plsc.subcore_barrier()

    def scat(j, kslot):
      return pltpu.make_async_copy(ones_v, acc_sh.at[dst_v.at[j]],
                                   sems[kslot])

    for kk in range(RING):
      scat(kk, kk).start(add=True)

    def body(t, carry):
      base = t * RING
      for kk in range(RING):
        scat(base + kk, kk).wait()
        scat(base + kk, kk).start(add=True)
      return carry

    lax.fori_loop(1, NBURST, body, 0)
    for kk in range(RING):
      scat(kk, kk).wait()
    plsc.subcore_barrier()
    pltpu.sync_copy(acc_sh.at[pl.ds(s * RPS, RPS)],
                    out_hbm.at[c, pl.ds(s * RPS, RPS)])

  return k


@functools.cache
def _make_sc_agg(feat, stage):
  mesh = plsc.VectorSubcoreMesh(core_axis_name="c", subcore_axis_name="s",
                                num_cores=NC, num_subcores=NS)

  @functools.partial(
      pl.kernel,
      out_type=jax.ShapeDtypeStruct((NC, NPAD, feat), jnp.float32),
      mesh=mesh,
      compiler_params=pltpu.CompilerParams(use_tc_tiling_on_sc=False),
      scratch_types=[
          pltpu.VMEM((CPW, CHUNK), jnp.int32),
          pltpu.VMEM((CPW, CHUNK), jnp.int32),
          pltpu.VMEM_SHARED((NPAD, feat), jnp.float32),
      ] + ([pltpu.VMEM_SHARED((NPAD, feat), jnp.float32)] if stage else [])
        + [pltpu.VMEM((CHUNK, feat), jnp.float32)] * RING
        + [pltpu.SemaphoreType.DMA] * (2 * RING),
  )
  def k(ht_hbm, src_hbm, dst_hbm, zeros_hbm, out_hbm,
        src_v, dst_v, acc_sh, *rest):
    if stage:
      ht_sh, *rest = rest
    rows = rest[:RING]
    gsem = rest[RING:2 * RING]
    ssem = rest[2 * RING:]
    c = lax.axis_index("c")
    s = lax.axis_index("s")
    w = c * NS + s
    if stage:
      pltpu.sync_copy(ht_hbm.at[pl.ds(s * RPS, RPS)],
                      ht_sh.at[pl.ds(s * RPS, RPS)])
      gsrc = ht_sh
    else:
      gsrc = ht_hbm
    pltpu.sync_copy(zeros_hbm.at[pl.ds(s * RPS, RPS)],
                    acc_sh.at[pl.ds(s * RPS, RPS)])
    pltpu.sync_copy(src_hbm.at[pl.ds(w * CPW, CPW)], src_v)
    pltpu.sync_copy(dst_hbm.at[pl.ds(w * CPW, CPW)], dst_v)
    plsc.subcore_barrier()

    def gat(j, kslot):
      return pltpu.make_async_copy(gsrc.at[src_v.at[j]], rows[kslot],
                                   gsem[kslot])

    def scat(j, kslot):
      return pltpu.make_async_copy(rows[kslot], acc_sh.at[dst_v.at[j]],
                                   ssem[kslot])

    for kk in range(RING):
      gat(kk, kk).start()
    for kk in range(RING):
      gat(kk, kk).wait()
      scat(kk, kk).start(add=True)

    def body(t, carry):
      base = t * RING
      for kk in range(RING):
        scat(base + kk, kk).wait()
        gat(base + kk, kk).start()
      for kk in range(RING):
        gat(base + kk, kk).wait()
        scat(base + kk, kk).start(add=True)
      return carry

    lax.fori_loop(1, NBURST, body, 0)
    for kk in range(RING):
      scat(kk, kk).wait()
    plsc.subcore_barrier()
    pltpu.sync_copy(acc_sh.at[pl.ds(s * RPS, RPS)],
                    out_hbm.at[c, pl.ds(s * RPS, RPS)])

  return k


def _row_mask(ncols):
  rows = lax.broadcasted_iota(jnp.int32, (NPAD, ncols), 0)
  return rows < NNODES


def _bn_relu(a, g, b, mask):
  am = jnp.where(mask, a, 0.0)
  mu = jnp.sum(am, axis=0, keepdims=True) / NNODES
  d = jnp.where(mask, a - mu, 0.0)
  var = jnp.sum(d * d, axis=0, keepdims=True) / NNODES
  h = (a - mu) * lax.rsqrt(var + EPS) * g + b
  return jnp.maximum(h, 0.0)


def _tc_pre_body(deg_ref, x_ref, w1_ref, ht1_ref, dinv_ref):
  deg = deg_ref[0] + deg_ref[1] + 1.0
  dinv = jnp.where(_row_mask(8), lax.rsqrt(deg), 0.0)
  dinv_ref[...] = dinv
  t1 = jnp.dot(x_ref[...], w1_ref[...], preferred_element_type=jnp.float32)
  ht1_ref[...] = dinv[:, 0:1] * t1


_TC_PRE = pl.pallas_call(
    _tc_pre_body,
    out_shape=(jax.ShapeDtypeStruct((NPAD, 32), jnp.float32),
               jax.ShapeDtypeStruct((NPAD, 8), jnp.float32)))


def _tc_mid1_body(acc_ref, ht_ref, dinv_ref, b_ref, g_ref, be_ref, out_ref):
  dinv = dinv_ref[...][:, 0:1]
  a = dinv * (acc_ref[0] + acc_ref[1] + ht_ref[...]) + b_ref[...]
  mask = _row_mask(a.shape[1])
  h = _bn_relu(a, g_ref[...], be_ref[...], mask)
  out_ref[...] = jnp.where(mask, dinv * h, 0.0)


_TC_MID1 = pl.pallas_call(
    _tc_mid1_body,
    out_shape=jax.ShapeDtypeStruct((NPAD, 32), jnp.float32))


def _tc_mid2_body(acc_ref, ht_ref, dinv_ref, w_ref, b_ref, g_ref, be_ref,
                  out_ref):
  dinv = dinv_ref[...][:, 0:1]
  u = dinv * (acc_ref[0] + acc_ref[1] + ht_ref[...])
  a = jnp.dot(u, w_ref[...], preferred_element_type=jnp.float32) + b_ref[...]
  mask = _row_mask(a.shape[1])
  h = _bn_relu(a, g_ref[...], be_ref[...], mask)
  out_ref[...] = jnp.where(mask, dinv * h, 0.0)


_TC_MID2 = pl.pallas_call(
    _tc_mid2_body,
    out_shape=jax.ShapeDtypeStruct((NPAD, 64), jnp.float32))


def _tc_fin_body(acca_ref, accb_ref, ht_ref, dinv_ref, w3_ref, b3_ref,
                 g3_ref, be3_ref, wf_ref, bf_ref, out_ref):
  dinv = dinv_ref[...][:, 0:1]
  agg = jnp.concatenate([acca_ref[0] + acca_ref[1],
                         accb_ref[0] + accb_ref[1]], axis=1)
  u = dinv * (agg + ht_ref[...])
  a = jnp.dot(u, w3_ref[...], preferred_element_type=jnp.float32) + b3_ref[...]
  mask = _row_mask(a.shape[1])
  h = _bn_relu(a, g3_ref[...], be3_ref[...], mask)
  logits = jnp.dot(h, wf_ref[...], preferred_element_type=jnp.float32)
  logits = logits + bf_ref[...]
  m = jnp.max(logits, axis=1, keepdims=True)
  lse = jnp.log(jnp.sum(jnp.exp(logits - m), axis=1, keepdims=True)) + m
  out_ref[...] = logits - lse


_TC_FIN = pl.pallas_call(
    _tc_fin_body,
    out_shape=jax.ShapeDtypeStruct((NPAD, 16), jnp.float32))


def kernel(x, edge_index, W1, b1, g1, be1, W2, b2, g2, be2, W3, b3, g3, be3,
           Wf, bf):
  src = edge_index[0]
  dst = edge_index[1]
  fill_src = jnp.full((EPAD - NEDGES,), NNODES, jnp.int32)
  fill_dst = NNODES + jnp.arange(EPAD - NEDGES, dtype=jnp.int32) % (
      NPAD - NNODES)
  src2 = jnp.concatenate([src, fill_src]).reshape(NW * CPW, CHUNK)
  dst2 = jnp.concatenate([dst, fill_dst]).reshape(NW * CPW, CHUNK)
  xp = jnp.pad(x, ((0, NPAD - NNODES), (0, 0)))
  z8 = jnp.zeros((NPAD, 8), jnp.float32)
  z32 = jnp.zeros((NPAD, 32), jnp.float32)
  ones8 = jnp.ones((CHUNK, 8), jnp.float32)
  wfp = jnp.pad(Wf, ((0, 0), (0, 6)))
  bfp = jnp.concatenate([bf, jnp.full((6,), -1e30, jnp.float32)])

  degp = _make_sc_degree()(dst2, ones8, z8)
  ht1, dinv8 = _TC_PRE(degp, xp, W1)
  acc1 = _make_sc_agg(32, True)(ht1, src2, dst2, z32)
  ht2 = _TC_MID1(acc1, ht1, dinv8, b1.reshape(1, -1), g1.reshape(1, -1),
                 be1.reshape(1, -1))
  acc2 = _make_sc_agg(32, True)(ht2, src2, dst2, z32)
  ht3 = _TC_MID2(acc2, ht2, dinv8, W2, b2.reshape(1, -1), g2.reshape(1, -1),
                 be2.reshape(1, -1))
  acc3a = _make_sc_agg(32, True)(ht3[:, :32], src2, dst2, z32)
  acc3b = _make_sc_agg(32, True)(ht3[:, 32:], src2, dst2, z32)
  outp = _TC_FIN(acc3a, acc3b, ht3, dinv8, W3, b3.reshape(1, -1),
                 g3.reshape(1, -1), be3.reshape(1, -1), wfp,
                 bfp.reshape(1, -1))
  return outp[:NNODES, :10]
